# baseline (speedup 1.0000x reference)
import jax
import jax.numpy as jnp
from jax import lax
from jax.experimental import pallas as pl
from jax.experimental.pallas import tpu as pltpu
from jax.experimental.pallas import tpu_sc as plsc

M = 1000000
B = 16384
ALPHA = 0.9

NC = 2
NS = 16
NW = NC * NS
L = 16

CH = 31256
CH_LAST = M - (NW - 1) * CH
assert CH % 8 == 0 and CH_LAST % 8 == 0 and CH_LAST <= CH
NB = B // L
NSTAG = 16
SEG = B // NSTAG
PCAP = B + L
UNROLL_A = 8


def _ema_body(data_hbm, idx_hbm, par_hbm, upd_hbm, out_hbm,
              pch, uch, idxv, datav, gidx, dcomp, semb, semc):
    wid = lax.axis_index("s") * NC + lax.axis_index("c")
    lo = wid * CH
    is_last = wid == NW - 1

    def batch_copies():
        cps = []
        for j in range(NSTAG):
            part = lax.rem(wid + j, NSTAG)
            off = part * SEG
            cps.append(pltpu.make_async_copy(
                idx_hbm.at[pl.ds(off, SEG)], idxv.at[pl.ds(off, SEG)], semb))
            cps.append(pltpu.make_async_copy(
                data_hbm.at[pl.ds(off, SEG)], datav.at[pl.ds(off, SEG)], semb))
        return cps

    def chunk_copies(n):
        src = par_hbm.at[pl.ds(lo, n)]
        return [
            pltpu.make_async_copy(src, pch.at[pl.ds(0, n)], semc),
            pltpu.make_async_copy(upd_hbm.at[pl.ds(lo, n)],
                                  uch.at[pl.ds(0, n)], semc),
        ]

    @pl.when(jnp.logical_not(is_last))
    def _():
        for c in chunk_copies(CH):
            c.start()

    @pl.when(is_last)
    def _():
        for c in chunk_copies(CH_LAST):
            c.start()

    for c in batch_copies():
        c.start()
    for c in batch_copies():
        c.wait()

    size_u = (jnp.where(is_last, CH_LAST, CH)).astype(jnp.uint32)
    lane = lax.iota(jnp.int32, L)

    def stepA(i, cursor):
        base = i * UNROLL_A * L
        for k in range(UNROLL_A):
            off = base + k * L
            idx = idxv[pl.ds(off, L)]
            d = datav[pl.ds(off, L)]
            m = (idx - lo).astype(jnp.uint32) < size_u
            plsc.store_compressed(gidx.at[pl.ds(cursor, L)], idx, mask=m)
            plsc.store_compressed(dcomp.at[pl.ds(cursor, L)], d, mask=m)
            cursor = cursor + plsc.all_reduce_population_count(m)[0]
        return cursor

    n = lax.fori_loop(0, NB // UNROLL_A, stepA, jnp.int32(0))
    nv_regs = (n + (L - 1)) // L

    @pl.when(jnp.logical_not(is_last))
    def _():
        for c in chunk_copies(CH):
            c.wait()

    @pl.when(is_last)
    def _():
        for c in chunk_copies(CH_LAST):
            c.wait()


    def gather_step(v, _):
        vo = v * L
        mB = (lane + vo) < n
        loc = gidx[pl.ds(vo, L)] - lo
        p = plsc.load_gather(pch, [loc], mask=mB)
        u = plsc.load_gather(uch, [loc], mask=mB)
        d = dcomp[pl.ds(vo, L)]
        dcomp[pl.ds(vo, L)] = ALPHA * p + (1.0 - ALPHA * u) * d
        return _

    lax.fori_loop(0, nv_regs, gather_step, None)

    def scatter_step(v, _):
        vo = v * L
        mB = (lane + vo) < n
        loc = gidx[pl.ds(vo, L)] - lo
        plsc.store_scatter(pch, [loc], dcomp[pl.ds(vo, L)], mask=mB)
        return _

    lax.fori_loop(0, nv_regs, scatter_step, None)

    @pl.when(jnp.logical_not(is_last))
    def _():
        pltpu.sync_copy(pch, out_hbm.at[pl.ds(lo, CH)])

    @pl.when(is_last)
    def _():
        pltpu.sync_copy(pch.at[pl.ds(0, CH_LAST)], out_hbm.at[pl.ds(lo, CH_LAST)])


@jax.jit
def _ema_update(data, index, parameter, updated):
    mesh = plsc.VectorSubcoreMesh(core_axis_name="c", subcore_axis_name="s",
                                  num_cores=NC, num_subcores=NS)
    return pl.kernel(
        _ema_body,
        out_type=jax.ShapeDtypeStruct((M,), jnp.float32),
        mesh=mesh,
        compiler_params=pltpu.CompilerParams(needs_layout_passes=False, skip_device_barrier=True),
        scratch_types=[
            pltpu.VMEM((CH,), jnp.float32),
            pltpu.VMEM((CH,), jnp.float32),
            pltpu.VMEM((B,), jnp.int32),
            pltpu.VMEM((B,), jnp.float32),
            pltpu.VMEM((PCAP,), jnp.int32),
            pltpu.VMEM((PCAP,), jnp.float32),
            pltpu.SemaphoreType.DMA,
            pltpu.SemaphoreType.DMA,
        ],
    )(data, index, parameter, updated)


def kernel(data, index, parameter, updated):
    return _ema_update(data, index, parameter, updated)


# speedup vs baseline: 1.0103x; 1.0103x over previous
import jax
import jax.numpy as jnp
from jax import lax
from jax.experimental import pallas as pl
from jax.experimental.pallas import tpu as pltpu
from jax.experimental.pallas import tpu_sc as plsc

M = 1000000
B = 16384
ALPHA = 0.9

NC = 2
NS = 16
NW = NC * NS
L = 16

CH = 31256
CH_LAST = M - (NW - 1) * CH
assert CH % 8 == 0 and CH_LAST % 8 == 0 and CH_LAST <= CH
NB = B // L
NSTAG = 8
SEG = B // NSTAG
PCAP = B + L
UNROLL_A = 8


def _ema_body(data_hbm, idx_hbm, par_hbm, upd_hbm, out_hbm,
              pch, uch, idxv, datav, gidx, dcomp, semb, semc):
    wid = lax.axis_index("s") * NC + lax.axis_index("c")
    lo = wid * CH
    is_last = wid == NW - 1

    def batch_copies():
        cps = []
        for j in range(NSTAG):
            part = lax.rem(wid + j, NSTAG)
            off = part * SEG
            cps.append(pltpu.make_async_copy(
                idx_hbm.at[pl.ds(off, SEG)], idxv.at[pl.ds(off, SEG)], semb))
            cps.append(pltpu.make_async_copy(
                data_hbm.at[pl.ds(off, SEG)], datav.at[pl.ds(off, SEG)], semb))
        return cps

    def chunk_copies(n):
        src = par_hbm.at[pl.ds(lo, n)]
        return [
            pltpu.make_async_copy(src, pch.at[pl.ds(0, n)], semc),
            pltpu.make_async_copy(upd_hbm.at[pl.ds(lo, n)],
                                  uch.at[pl.ds(0, n)], semc),
        ]

    @pl.when(jnp.logical_not(is_last))
    def _():
        for c in chunk_copies(CH):
            c.start()

    @pl.when(is_last)
    def _():
        for c in chunk_copies(CH_LAST):
            c.start()

    for c in batch_copies():
        c.start()
    for c in batch_copies():
        c.wait()

    size_u = (jnp.where(is_last, CH_LAST, CH)).astype(jnp.uint32)
    lane = lax.iota(jnp.int32, L)

    def stepA(i, cursor):
        base = i * UNROLL_A * L
        for k in range(UNROLL_A):
            off = base + k * L
            idx = idxv[pl.ds(off, L)]
            d = datav[pl.ds(off, L)]
            m = (idx - lo).astype(jnp.uint32) < size_u
            plsc.store_compressed(gidx.at[pl.ds(cursor, L)], idx, mask=m)
            plsc.store_compressed(dcomp.at[pl.ds(cursor, L)], d, mask=m)
            cursor = cursor + plsc.all_reduce_population_count(m)[0]
        return cursor

    n = lax.fori_loop(0, NB // UNROLL_A, stepA, jnp.int32(0))
    nv_regs = (n + (L - 1)) // L

    @pl.when(jnp.logical_not(is_last))
    def _():
        for c in chunk_copies(CH):
            c.wait()

    @pl.when(is_last)
    def _():
        for c in chunk_copies(CH_LAST):
            c.wait()


    def gather_step(v, _):
        vo = v * L
        mB = (lane + vo) < n
        loc = gidx[pl.ds(vo, L)] - lo
        p = plsc.load_gather(pch, [loc], mask=mB)
        u = plsc.load_gather(uch, [loc], mask=mB)
        d = dcomp[pl.ds(vo, L)]
        dcomp[pl.ds(vo, L)] = ALPHA * p + (1.0 - ALPHA * u) * d
        return _

    lax.fori_loop(0, nv_regs, gather_step, None)

    def scatter_step(v, _):
        vo = v * L
        mB = (lane + vo) < n
        loc = gidx[pl.ds(vo, L)] - lo
        plsc.store_scatter(pch, [loc], dcomp[pl.ds(vo, L)], mask=mB)
        return _

    lax.fori_loop(0, nv_regs, scatter_step, None)

    @pl.when(jnp.logical_not(is_last))
    def _():
        pltpu.sync_copy(pch, out_hbm.at[pl.ds(lo, CH)])

    @pl.when(is_last)
    def _():
        pltpu.sync_copy(pch.at[pl.ds(0, CH_LAST)], out_hbm.at[pl.ds(lo, CH_LAST)])


@jax.jit
def _ema_update(data, index, parameter, updated):
    mesh = plsc.VectorSubcoreMesh(core_axis_name="c", subcore_axis_name="s",
                                  num_cores=NC, num_subcores=NS)
    return pl.kernel(
        _ema_body,
        out_type=jax.ShapeDtypeStruct((M,), jnp.float32),
        mesh=mesh,
        compiler_params=pltpu.CompilerParams(needs_layout_passes=False, skip_device_barrier=True),
        scratch_types=[
            pltpu.VMEM((CH,), jnp.float32),
            pltpu.VMEM((CH,), jnp.float32),
            pltpu.VMEM((B,), jnp.int32),
            pltpu.VMEM((B,), jnp.float32),
            pltpu.VMEM((PCAP,), jnp.int32),
            pltpu.VMEM((PCAP,), jnp.float32),
            pltpu.SemaphoreType.DMA,
            pltpu.SemaphoreType.DMA,
        ],
    )(data, index, parameter, updated)


def kernel(data, index, parameter, updated):
    return _ema_update(data, index, parameter, updated)
